# baseline (device time: 15029 ns/iter reference)
import jax
import jax.numpy as jnp
from jax import lax
from jax.experimental import pallas as pl
from jax.experimental.pallas import tpu as pltpu

N_DEV = 4
EPS = 1e-5
GLOBAL_H = 256


def kernel(x, Wp):
    b, h_loc, w, c = x.shape
    c_out = Wp.shape[1]
    count = GLOBAL_H * w

    def body(x_ref, wp_ref, out_ref, comm_ref, send_sems, recv_sems):
        my = lax.axis_index("i")
        left = (my + N_DEV - 1) % N_DEV
        right = (my + 1) % N_DEV

        barrier_sem = pltpu.get_barrier_semaphore()
        for nbr in (left, right):
            pl.semaphore_signal(
                barrier_sem, inc=1,
                device_id=(nbr,), device_id_type=pl.DeviceIdType.MESH,
            )
        pl.semaphore_wait(barrier_sem, 2)

        xf = x_ref[...]
        s1 = jnp.sum(xf, axis=(1, 2))
        s2 = jnp.sum(xf * xf, axis=(1, 2))
        comm_ref[0, 0] = s1
        comm_ref[0, 1] = s2

        for hop in range(N_DEV - 1):
            rdma = pltpu.make_async_remote_copy(
                src_ref=comm_ref.at[hop],
                dst_ref=comm_ref.at[hop + 1],
                send_sem=send_sems.at[hop],
                recv_sem=recv_sems.at[hop],
                device_id=(right,),
                device_id_type=pl.DeviceIdType.MESH,
            )
            rdma.start()
            rdma.wait()

        totals = jnp.sum(comm_ref[...], axis=0)
        mean = totals[0] / count
        var = totals[1] / count - mean * mean
        inv = lax.rsqrt(var + EPS)

        xn = (xf - mean[:, None, None, :]) * inv[:, None, None, :]
        a = xn / (1.0 + jnp.exp(-xn))

        a16 = a.astype(jnp.bfloat16).reshape(b * h_loc * w, c)
        wp16 = wp_ref[...].astype(jnp.bfloat16)
        res = jnp.dot(a16, wp16, preferred_element_type=jnp.float32)
        out_ref[...] = res.reshape(b, h_loc, w, c_out).astype(out_ref.dtype)

    return pl.pallas_call(
        body,
        out_shape=jax.ShapeDtypeStruct((b, h_loc, w, c_out), jnp.bfloat16),
        in_specs=[
            pl.BlockSpec(memory_space=pltpu.VMEM),
            pl.BlockSpec(memory_space=pltpu.VMEM),
        ],
        out_specs=pl.BlockSpec(memory_space=pltpu.VMEM),
        scratch_shapes=[
            pltpu.VMEM((N_DEV, 2, b, c), jnp.float32),
            pltpu.SemaphoreType.DMA((N_DEV - 1,)),
            pltpu.SemaphoreType.DMA((N_DEV - 1,)),
        ],
        compiler_params=pltpu.CompilerParams(collective_id=0),
    )(x, Wp)


# device time: 13228 ns/iter; 1.1362x vs baseline; 1.1362x over previous
import jax
import jax.numpy as jnp
from jax import lax
from jax.experimental import pallas as pl
from jax.experimental.pallas import tpu as pltpu

N_DEV = 4
EPS = 1e-5
GLOBAL_H = 256


def kernel(x, Wp):
    b, h_loc, w, c = x.shape
    c_out = Wp.shape[1]
    count = GLOBAL_H * w

    def body(x_ref, wp_ref, out_ref, comm_ref, send_sems, recv_sems):
        my = lax.axis_index("i")
        p1 = my ^ 1
        p2 = 3 - my

        barrier_sem = pltpu.get_barrier_semaphore()
        for nbr in (p1, p2):
            pl.semaphore_signal(
                barrier_sem, inc=1,
                device_id=(nbr,), device_id_type=pl.DeviceIdType.MESH,
            )
        pl.semaphore_wait(barrier_sem, 2)

        xf = x_ref[...]
        s1 = jnp.sum(xf, axis=(1, 2))
        s2 = jnp.sum(xf * xf, axis=(1, 2))
        comm_ref[0, 0] = s1
        comm_ref[0, 1] = s2

        rdma1 = pltpu.make_async_remote_copy(
            src_ref=comm_ref.at[0],
            dst_ref=comm_ref.at[1],
            send_sem=send_sems.at[0],
            recv_sem=recv_sems.at[0],
            device_id=(p1,),
            device_id_type=pl.DeviceIdType.MESH,
        )
        rdma1.start()

        xb = xf.astype(jnp.bfloat16)
        wp16 = wp_ref[...].astype(jnp.bfloat16)

        rdma1.wait()
        comm_ref[2] = comm_ref[0] + comm_ref[1]

        rdma2 = pltpu.make_async_remote_copy(
            src_ref=comm_ref.at[2],
            dst_ref=comm_ref.at[3],
            send_sem=send_sems.at[1],
            recv_sem=recv_sems.at[1],
            device_id=(p2,),
            device_id_type=pl.DeviceIdType.MESH,
        )
        rdma2.start()
        rdma2.wait()

        totals = comm_ref[2] + comm_ref[3]
        mean = totals[0] / count
        var = totals[1] / count - mean * mean
        inv = lax.rsqrt(var + EPS)

        m16 = mean.astype(jnp.bfloat16)
        i16 = inv.astype(jnp.bfloat16)
        xn = (xb - m16[:, None, None, :]) * i16[:, None, None, :]
        a = xn / (1.0 + jnp.exp(-xn))

        res = jnp.dot(
            a.reshape(b * h_loc * w, c), wp16,
            preferred_element_type=jnp.float32,
        )
        out_ref[...] = res.reshape(b, h_loc, w, c_out).astype(out_ref.dtype)

    return pl.pallas_call(
        body,
        out_shape=jax.ShapeDtypeStruct((b, h_loc, w, c_out), jnp.bfloat16),
        in_specs=[
            pl.BlockSpec(memory_space=pltpu.VMEM),
            pl.BlockSpec(memory_space=pltpu.VMEM),
        ],
        out_specs=pl.BlockSpec(memory_space=pltpu.VMEM),
        scratch_shapes=[
            pltpu.VMEM((4, 2, b, c), jnp.float32),
            pltpu.SemaphoreType.DMA((2,)),
            pltpu.SemaphoreType.DMA((2,)),
        ],
        compiler_params=pltpu.CompilerParams(collective_id=0),
    )(x, Wp)


# device time: 12034 ns/iter; 1.2489x vs baseline; 1.0992x over previous
import jax
import jax.numpy as jnp
from jax import lax
from jax.experimental import pallas as pl
from jax.experimental.pallas import tpu as pltpu

N_DEV = 4
EPS = 1e-5
GLOBAL_H = 256


def kernel(x, Wp):
    b, h_loc, w, c = x.shape
    c_out = Wp.shape[1]
    count = GLOBAL_H * w

    def body(x_ref, wp_ref, out_ref, comm_ref, send_sems, recv_sems):
        my = lax.axis_index("i")

        barrier_sem = pltpu.get_barrier_semaphore()
        for m in (1, 2, 3):
            pl.semaphore_signal(
                barrier_sem, inc=1,
                device_id=(my ^ m,), device_id_type=pl.DeviceIdType.MESH,
            )
        pl.semaphore_wait(barrier_sem, 3)

        def push(j, rdma_list):
            for m in (1, 2, 3):
                r = pltpu.make_async_remote_copy(
                    src_ref=comm_ref.at[0, j],
                    dst_ref=comm_ref.at[m, j],
                    send_sem=send_sems.at[m - 1, j],
                    recv_sem=recv_sems.at[m - 1, j],
                    device_id=(my ^ m,),
                    device_id_type=pl.DeviceIdType.MESH,
                )
                r.start()
                rdma_list.append(r)

        rdmas = []
        xf = x_ref[...]

        s1 = jnp.sum(xf, axis=(1, 2))
        comm_ref[0, 0] = s1
        push(0, rdmas)

        s2 = jnp.sum(xf * xf, axis=(1, 2))
        comm_ref[0, 1] = s2
        push(1, rdmas)

        xb = xf.astype(jnp.bfloat16)
        wp16 = wp_ref[...].astype(jnp.bfloat16)

        for r in rdmas:
            r.wait()

        totals = (
            comm_ref[0] + comm_ref[1] + comm_ref[2] + comm_ref[3]
        )
        mean = totals[0] / count
        var = totals[1] / count - mean * mean
        inv = lax.rsqrt(var + EPS)

        m16 = mean.astype(jnp.bfloat16)
        i16 = inv.astype(jnp.bfloat16)
        xn = (xb - m16[:, None, None, :]) * i16[:, None, None, :]
        a = xn / (1.0 + jnp.exp(-xn))

        res = jnp.dot(
            a.reshape(b * h_loc * w, c), wp16,
            preferred_element_type=jnp.float32,
        )
        out_ref[...] = res.reshape(b, h_loc, w, c_out).astype(out_ref.dtype)

    return pl.pallas_call(
        body,
        out_shape=jax.ShapeDtypeStruct((b, h_loc, w, c_out), jnp.bfloat16),
        in_specs=[
            pl.BlockSpec(memory_space=pltpu.VMEM),
            pl.BlockSpec(memory_space=pltpu.VMEM),
        ],
        out_specs=pl.BlockSpec(memory_space=pltpu.VMEM),
        scratch_shapes=[
            pltpu.VMEM((N_DEV, 2, b, c), jnp.float32),
            pltpu.SemaphoreType.DMA((3, 2)),
            pltpu.SemaphoreType.DMA((3, 2)),
        ],
        compiler_params=pltpu.CompilerParams(collective_id=0),
    )(x, Wp)
